# baseline (speedup 1.0000x reference)
import jax
import jax.numpy as jnp
from jax import lax
from jax.experimental import pallas as pl
from jax.experimental.pallas import tpu as pltpu
from jax.experimental.pallas import tpu_sc as plsc

D_MODEL = 128
BATCH = 16384
LANES = 16

_info = plsc.get_sparse_core_info()
NUM_CORES = _info.num_cores
NUM_SUBCORES = _info.num_subcores
NW = NUM_CORES * NUM_SUBCORES
BPW = BATCH // NW
CHUNK = 128
NCHUNK = BPW // CHUNK


def _body(x_hbm, t_hbm, pe_hbm, out_hbm,
          idx_v, x_big, pe_v0, pe_v1, gsem0, gsem1, xsem0, xsem1, osem):
    wid = lax.axis_index("s") * NUM_CORES + lax.axis_index("c")
    base = wid * BPW
    pltpu.sync_copy(t_hbm.at[pl.ds(base, BPW)], idx_v)

    pe_bufs = (pe_v0, pe_v1)
    gsems = (gsem0, gsem1)
    xsems = (xsem0, xsem1)

    def fire(ci, pe_b, gsem, xsem):
        pltpu.async_copy(
            pe_hbm.at[idx_v.at[pl.ds(ci * CHUNK, CHUNK)]], pe_b, gsem)
        pltpu.async_copy(
            x_hbm.at[pl.ds(base + ci * CHUNK, CHUNK)],
            x_big.at[pl.ds(ci * CHUNK, CHUNK)], xsem)

    fire(0, pe_v0, gsem0, xsem0)
    fire(1, pe_v1, gsem1, xsem1)

    def pair(pi, carry):
        for k in range(2):
            ci = 2 * pi + k
            pe_b = pe_bufs[k]
            pltpu.make_async_copy(
                pe_hbm.at[idx_v.at[pl.ds(0, CHUNK)]], pe_b, gsems[k]).wait()
            pltpu.make_async_copy(
                x_hbm.at[pl.ds(0, CHUNK)],
                x_big.at[pl.ds(0, CHUNK)], xsems[k]).wait()

            def row(r, c2):
                xr = ci * CHUNK + r
                for j in range(D_MODEL // LANES):
                    sl = pl.ds(j * LANES, LANES)
                    x_big[xr, sl] = x_big[xr, sl] + pe_b[r, sl]
                return c2

            lax.fori_loop(0, CHUNK, row, 0)
            pltpu.async_copy(
                x_big.at[pl.ds(ci * CHUNK, CHUNK)],
                out_hbm.at[pl.ds(base + ci * CHUNK, CHUNK)], osem)

            @pl.when(ci + 2 < NCHUNK)
            def _():
                fire(ci + 2, pe_b, gsems[k], xsems[k])
        return carry

    lax.fori_loop(0, NCHUNK // 2, pair, 0)
    for _ in range(NCHUNK):
        pltpu.make_async_copy(
            x_big.at[pl.ds(0, CHUNK)],
            out_hbm.at[pl.ds(0, CHUNK)], osem).wait()


@jax.jit
def _run(x, t, pe2d):
    mesh = plsc.VectorSubcoreMesh(core_axis_name="c", subcore_axis_name="s")
    k = pl.kernel(
        _body,
        mesh=mesh,
        out_type=jax.ShapeDtypeStruct((BATCH, D_MODEL), jnp.float32),
        scratch_types=[
            pltpu.VMEM((BPW,), jnp.int32),
            pltpu.VMEM((BPW, D_MODEL), jnp.float32),
            pltpu.VMEM((CHUNK, D_MODEL), jnp.float32),
            pltpu.VMEM((CHUNK, D_MODEL), jnp.float32),
            pltpu.SemaphoreType.DMA,
            pltpu.SemaphoreType.DMA,
            pltpu.SemaphoreType.DMA,
            pltpu.SemaphoreType.DMA,
            pltpu.SemaphoreType.DMA,
        ],
    )
    return k(x, t, pe2d)


def kernel(x, t, pe):
    out = _run(x, t.astype(jnp.int32), pe.reshape(pe.shape[1], pe.shape[2]))
    return out[None]


# speedup vs baseline: 1.4364x; 1.4364x over previous
import jax
import jax.numpy as jnp
from jax import lax
from jax.experimental import pallas as pl
from jax.experimental.pallas import tpu as pltpu
from jax.experimental.pallas import tpu_sc as plsc

D_MODEL = 128
BATCH = 16384
LANES = 16

_info = plsc.get_sparse_core_info()
NUM_CORES = _info.num_cores
NUM_SUBCORES = _info.num_subcores
NW = NUM_CORES * NUM_SUBCORES
BPW = BATCH // NW
CHUNK = 128
NCHUNK = BPW // CHUNK
DEPTH = 3


def _body(x_hbm, t_hbm, pe_hbm, out_hbm,
          idx_v, x_big, pe_v0, pe_v1, pe_v2,
          gsem0, gsem1, gsem2, xsem, osem):
    wid = lax.axis_index("s") * NUM_CORES + lax.axis_index("c")
    base = wid * BPW
    pltpu.sync_copy(t_hbm.at[pl.ds(base, BPW)], idx_v)
    cx = pltpu.async_copy(x_hbm.at[pl.ds(base, BPW)], x_big, xsem)

    pe_bufs = (pe_v0, pe_v1, pe_v2)
    gsems = (gsem0, gsem1, gsem2)
    copies = [None] * NCHUNK
    for ci in range(DEPTH):
        copies[ci] = pltpu.async_copy(
            pe_hbm.at[idx_v.at[pl.ds(ci * CHUNK, CHUNK)]],
            pe_bufs[ci], gsems[ci])
    cx.wait()

    stores = []
    for ci in range(NCHUNK):
        k = ci % DEPTH
        copies[ci].wait()
        pe_b = pe_bufs[k]

        @plsc.parallel_loop(0, CHUNK, unroll=2)
        def _row(r):
            xr = ci * CHUNK + r
            for j in range(D_MODEL // LANES):
                sl = pl.ds(j * LANES, LANES)
                plsc.addupdate(x_big.at[xr, sl], pe_b[r, sl])

        if ci + DEPTH < NCHUNK:
            copies[ci + DEPTH] = pltpu.async_copy(
                pe_hbm.at[idx_v.at[pl.ds((ci + DEPTH) * CHUNK, CHUNK)]],
                pe_b, gsems[k])
        stores.append(pltpu.async_copy(
            x_big.at[pl.ds(ci * CHUNK, CHUNK)],
            out_hbm.at[pl.ds(base + ci * CHUNK, CHUNK)], osem))
    for s in stores:
        s.wait()


@jax.jit
def _run(x, t, pe2d):
    mesh = plsc.VectorSubcoreMesh(core_axis_name="c", subcore_axis_name="s")
    k = pl.kernel(
        _body,
        mesh=mesh,
        out_type=jax.ShapeDtypeStruct((BATCH, D_MODEL), jnp.float32),
        scratch_types=[
            pltpu.VMEM((BPW,), jnp.int32),
            pltpu.VMEM((BPW, D_MODEL), jnp.float32),
            pltpu.VMEM((CHUNK, D_MODEL), jnp.float32),
            pltpu.VMEM((CHUNK, D_MODEL), jnp.float32),
            pltpu.VMEM((CHUNK, D_MODEL), jnp.float32),
            pltpu.SemaphoreType.DMA,
            pltpu.SemaphoreType.DMA,
            pltpu.SemaphoreType.DMA,
            pltpu.SemaphoreType.DMA,
            pltpu.SemaphoreType.DMA,
        ],
    )
    return k(x, t, pe2d)


def kernel(x, t, pe):
    out = _run(x, t.astype(jnp.int32), pe.reshape(pe.shape[1], pe.shape[2]))
    return out[None]


# speedup vs baseline: 1.5135x; 1.0537x over previous
import jax
import jax.numpy as jnp
from jax import lax
from jax.experimental import pallas as pl
from jax.experimental.pallas import tpu as pltpu
from jax.experimental.pallas import tpu_sc as plsc

D_MODEL = 128
BATCH = 16384
LANES = 16

_info = plsc.get_sparse_core_info()
NUM_CORES = _info.num_cores
NUM_SUBCORES = _info.num_subcores
NW = NUM_CORES * NUM_SUBCORES
BPW = BATCH // NW
CHUNK = 128
NCHUNK = BPW // CHUNK
DEPTH = 3


def _body(x_hbm, t_hbm, pe_hbm, out_hbm,
          idx_v, x_big, pe_v0, pe_v1, pe_v2,
          gsem0, gsem1, gsem2, xsem0, xsem1, xsem2, xsem3, osem):
    wid = lax.axis_index("s") * NUM_CORES + lax.axis_index("c")
    base = wid * BPW
    pltpu.sync_copy(t_hbm.at[pl.ds(base, BPW)], idx_v)

    pe_bufs = (pe_v0, pe_v1, pe_v2)
    gsems = (gsem0, gsem1, gsem2)
    xsems = (xsem0, xsem1, xsem2, xsem3)
    copies = [None] * NCHUNK
    xcopies = [None] * NCHUNK
    for ci in range(NCHUNK):
        if ci < DEPTH:
            copies[ci] = pltpu.async_copy(
                pe_hbm.at[idx_v.at[pl.ds(ci * CHUNK, CHUNK)]],
                pe_bufs[ci], gsems[ci])
        xcopies[ci] = pltpu.async_copy(
            x_hbm.at[pl.ds(base + ci * CHUNK, CHUNK)],
            x_big.at[pl.ds(ci * CHUNK, CHUNK)], xsems[ci])

    stores = []
    for ci in range(NCHUNK):
        k = ci % DEPTH
        copies[ci].wait()
        xcopies[ci].wait()
        pe_b = pe_bufs[k]

        @plsc.parallel_loop(0, CHUNK, unroll=2)
        def _row(r):
            xr = ci * CHUNK + r
            for j in range(D_MODEL // LANES):
                sl = pl.ds(j * LANES, LANES)
                plsc.addupdate(x_big.at[xr, sl], pe_b[r, sl])

        if ci + DEPTH < NCHUNK:
            copies[ci + DEPTH] = pltpu.async_copy(
                pe_hbm.at[idx_v.at[pl.ds((ci + DEPTH) * CHUNK, CHUNK)]],
                pe_b, gsems[k])
        stores.append(pltpu.async_copy(
            x_big.at[pl.ds(ci * CHUNK, CHUNK)],
            out_hbm.at[pl.ds(base + ci * CHUNK, CHUNK)], osem))
    for s in stores:
        s.wait()


@jax.jit
def _run(x, t, pe2d):
    mesh = plsc.VectorSubcoreMesh(core_axis_name="c", subcore_axis_name="s")
    k = pl.kernel(
        _body,
        mesh=mesh,
        out_type=jax.ShapeDtypeStruct((BATCH, D_MODEL), jnp.float32),
        scratch_types=[
            pltpu.VMEM((BPW,), jnp.int32),
            pltpu.VMEM((BPW, D_MODEL), jnp.float32),
            pltpu.VMEM((CHUNK, D_MODEL), jnp.float32),
            pltpu.VMEM((CHUNK, D_MODEL), jnp.float32),
            pltpu.VMEM((CHUNK, D_MODEL), jnp.float32),
            pltpu.SemaphoreType.DMA,
            pltpu.SemaphoreType.DMA,
            pltpu.SemaphoreType.DMA,
            pltpu.SemaphoreType.DMA,
            pltpu.SemaphoreType.DMA,
            pltpu.SemaphoreType.DMA,
            pltpu.SemaphoreType.DMA,
            pltpu.SemaphoreType.DMA,
        ],
    )
    return k(x, t, pe2d)


def kernel(x, t, pe):
    out = _run(x, t.astype(jnp.int32), pe.reshape(pe.shape[1], pe.shape[2]))
    return out[None]
